# baseline (speedup 1.0000x reference)
import jax
import jax.numpy as jnp
from jax import lax
from jax.experimental import pallas as pl
from jax.experimental.pallas import tpu as pltpu
from jax.experimental.pallas import tpu_sc as plsc

N = 16777216
NBINS = 10
NC, NS, L = 2, 16, 16
NW = NC * NS
PER_W = N // NW
CHUNK = 32768
NCHUNK = PER_W // CHUNK
UNROLL = 8
HIST = 2 * NBINS * L

_mesh = plsc.VectorSubcoreMesh(core_axis_name="c", subcore_axis_name="s")


def _hist_body(pred_hbm, target_hbm, out_hbm, buf0, buf1, hist, sem0, sem1):
    wid = lax.axis_index("s") * NC + lax.axis_index("c")
    base = wid * PER_W
    lane = lax.iota(jnp.int32, L)
    ones = jnp.ones((L,), jnp.float32)
    zeros = jnp.zeros((L,), jnp.float32)
    sems = (sem0, sem1)
    bufs = (buf0, buf1)

    for i in range(2 * NBINS):
        hist[pl.ds(i * L, L)] = zeros

    for a, src in enumerate((pred_hbm, target_hbm)):
        lane_a = lane * (2 * NBINS) + a * NBINS

        def start(c):
            return pltpu.async_copy(
                src.at[pl.ds(base + c * CHUNK, CHUNK)],
                bufs[c % 2],
                sems[c % 2],
            )

        copies = [None, None]
        copies[0] = start(0)
        for c in range(NCHUNK):
            if c + 1 < NCHUNK:
                copies[(c + 1) % 2] = start(c + 1)
            copies[c % 2].wait()
            bref = bufs[c % 2]

            @plsc.parallel_loop(0, CHUNK, step=L, unroll=UNROLL)
            def _loop(i):
                x = bref[pl.ds(i, L)]
                idx = (x * jnp.float32(NBINS)).astype(jnp.int32)
                idx = jnp.minimum(idx, NBINS - 1)
                plsc.addupdate_scatter(hist, [idx + lane_a], ones)

    pltpu.sync_copy(hist, out_hbm.at[pl.ds(wid * HIST, HIST)])


_hist_call = pl.kernel(
    _hist_body,
    out_type=jax.ShapeDtypeStruct((NW * HIST,), jnp.float32),
    mesh=_mesh,
    scratch_types=[
        pltpu.VMEM((CHUNK,), jnp.float32),
        pltpu.VMEM((CHUNK,), jnp.float32),
        pltpu.VMEM((HIST,), jnp.float32),
        pltpu.SemaphoreType.DMA,
        pltpu.SemaphoreType.DMA,
    ],
    compiler_params=pltpu.CompilerParams(needs_layout_passes=False),
)


def _red_body(h_ref, o_ref):
    h = h_ref[...]
    col = jnp.sum(h, axis=0, keepdims=True)
    gid = lax.broadcasted_iota(jnp.int32, (2 * NBINS, HIST), 1) % (2 * NBINS)
    bid = lax.broadcasted_iota(jnp.int32, (2 * NBINS, HIST), 0)
    colb = jnp.broadcast_to(col, (2 * NBINS, HIST))
    masked = jnp.where(gid == bid, colb, 0.0)
    o_ref[...] = jnp.sum(masked, axis=1, keepdims=True)


_red_call = pl.pallas_call(
    _red_body,
    out_shape=jax.ShapeDtypeStruct((2 * NBINS, 1), jnp.float32),
)


def kernel(pred, target):
    parts = _hist_call(pred, target)
    h = jnp.reshape(parts, (NW, HIST))
    counts = _red_call(h)[:, 0]
    p = counts[0:NBINS]
    t = counts[NBINS:2 * NBINS]
    p = p / p.sum()
    t = t / t.sum()
    p = p + 1e-08
    t = t + 1e-08
    return jnp.sum(t * (jnp.log(t) - jnp.log(p))) / NBINS


# speedup vs baseline: 1.2009x; 1.2009x over previous
import jax
import jax.numpy as jnp
from jax import lax
from jax.experimental import pallas as pl
from jax.experimental.pallas import tpu as pltpu
from jax.experimental.pallas import tpu_sc as plsc

N = 16777216
NBINS = 10
NC, NS, L = 2, 16, 16
NW = NC * NS
PER_W = N // NW
CHUNK = 32768
NCHUNK = PER_W // CHUNK
UNROLL = 8
HIST = 2 * NBINS * L

_mesh = plsc.VectorSubcoreMesh(core_axis_name="c", subcore_axis_name="s")


def _hist_body(pred_hbm, target_hbm, out_hbm, buf0, buf1, hist, sem0, sem1):
    wid = lax.axis_index("s") * NC + lax.axis_index("c")
    base = wid * PER_W
    lane = lax.iota(jnp.int32, L)
    ones = jnp.ones((L,), jnp.float32)
    zeros = jnp.zeros((L,), jnp.float32)
    sems = (sem0, sem1)
    bufs = (buf0, buf1)

    for i in range(2 * NBINS):
        hist[pl.ds(i * L, L)] = zeros

    for a, src in enumerate((pred_hbm, target_hbm)):
        lane_a = lane + a * NBINS * L

        def start(c):
            return pltpu.async_copy(
                src.at[pl.ds(base + c * CHUNK, CHUNK)],
                bufs[c % 2],
                sems[c % 2],
            )

        copies = [None, None]
        copies[0] = start(0)
        for c in range(NCHUNK):
            if c + 1 < NCHUNK:
                copies[(c + 1) % 2] = start(c + 1)
            copies[c % 2].wait()
            bref = bufs[c % 2]

            @plsc.parallel_loop(0, CHUNK, step=L, unroll=UNROLL)
            def _loop(i):
                x = bref[pl.ds(i, L)]
                idx16 = (x * jnp.float32(NBINS * L)).astype(jnp.int32) & ~(L - 1)
                idx16 = jnp.minimum(idx16, (NBINS - 1) * L)
                plsc.addupdate_scatter(hist, [idx16 + lane_a], ones)

    pltpu.sync_copy(hist, out_hbm.at[pl.ds(wid * HIST, HIST)])


_hist_call = pl.kernel(
    _hist_body,
    out_type=jax.ShapeDtypeStruct((NW * HIST,), jnp.float32),
    mesh=_mesh,
    scratch_types=[
        pltpu.VMEM((CHUNK,), jnp.float32),
        pltpu.VMEM((CHUNK,), jnp.float32),
        pltpu.VMEM((HIST,), jnp.float32),
        pltpu.SemaphoreType.DMA,
        pltpu.SemaphoreType.DMA,
    ],
    compiler_params=pltpu.CompilerParams(needs_layout_passes=False),
)


def _red_body(h_ref, o_ref):
    h = h_ref[...]
    col = jnp.sum(h, axis=0, keepdims=True)
    gid = lax.broadcasted_iota(jnp.int32, (2 * NBINS, HIST), 1) // L
    bid = lax.broadcasted_iota(jnp.int32, (2 * NBINS, HIST), 0)
    colb = jnp.broadcast_to(col, (2 * NBINS, HIST))
    masked = jnp.where(gid == bid, colb, 0.0)
    o_ref[...] = jnp.sum(masked, axis=1, keepdims=True)


_red_call = pl.pallas_call(
    _red_body,
    out_shape=jax.ShapeDtypeStruct((2 * NBINS, 1), jnp.float32),
)


def kernel(pred, target):
    parts = _hist_call(pred, target)
    h = jnp.reshape(parts, (NW, HIST))
    counts = _red_call(h)[:, 0]
    p = counts[0:NBINS]
    t = counts[NBINS:2 * NBINS]
    p = p / p.sum()
    t = t / t.sum()
    p = p + 1e-08
    t = t + 1e-08
    return jnp.sum(t * (jnp.log(t) - jnp.log(p))) / NBINS


# speedup vs baseline: 1.3434x; 1.1186x over previous
import jax
import jax.numpy as jnp
from jax import lax
from jax.experimental import pallas as pl
from jax.experimental.pallas import tpu as pltpu
from jax.experimental.pallas import tpu_sc as plsc

N = 16777216
NBINS = 10
NC, NS, L = 2, 16, 16
NW = NC * NS
PER_W = N // NW
CHUNK = 32768
NCHUNK = PER_W // CHUNK
UNROLL = 16
HIST = 2 * NBINS * L

_mesh = plsc.VectorSubcoreMesh(core_axis_name="c", subcore_axis_name="s")


def _hist_body(pred_hbm, target_hbm, out_hbm, buf0, buf1, hist, sem0, sem1):
    wid = lax.axis_index("s") * NC + lax.axis_index("c")
    base = wid * PER_W
    lane = lax.iota(jnp.int32, L)
    ones = jnp.ones((L,), jnp.float32)
    zeros = jnp.zeros((L,), jnp.float32)
    sems = (sem0, sem1)
    bufs = (buf0, buf1)

    for i in range(2 * NBINS):
        hist[pl.ds(i * L, L)] = zeros

    for a, src in enumerate((pred_hbm, target_hbm)):
        lane_a = lane + a * NBINS * L

        def start(c):
            return pltpu.async_copy(
                src.at[pl.ds(base + c * CHUNK, CHUNK)],
                bufs[c % 2],
                sems[c % 2],
            )

        copies = [None, None]
        copies[0] = start(0)
        for c in range(NCHUNK):
            if c + 1 < NCHUNK:
                copies[(c + 1) % 2] = start(c + 1)
            copies[c % 2].wait()
            bref = bufs[c % 2]

            @plsc.parallel_loop(0, CHUNK, step=L, unroll=UNROLL)
            def _loop(i):
                x = bref[pl.ds(i, L)]
                y = jnp.minimum(x * jnp.float32(NBINS * L), jnp.float32(NBINS * L - 1))
                idx16 = y.astype(jnp.int32) & ~(L - 1)
                plsc.addupdate_scatter(hist, [idx16 + lane_a], ones)

    pltpu.sync_copy(hist, out_hbm.at[pl.ds(wid * HIST, HIST)])


_hist_call = pl.kernel(
    _hist_body,
    out_type=jax.ShapeDtypeStruct((NW * HIST,), jnp.float32),
    mesh=_mesh,
    scratch_types=[
        pltpu.VMEM((CHUNK,), jnp.float32),
        pltpu.VMEM((CHUNK,), jnp.float32),
        pltpu.VMEM((HIST,), jnp.float32),
        pltpu.SemaphoreType.DMA,
        pltpu.SemaphoreType.DMA,
    ],
    compiler_params=pltpu.CompilerParams(needs_layout_passes=False),
)


def _red_body(h_ref, o_ref):
    h = h_ref[...]
    col = jnp.sum(h, axis=0, keepdims=True)
    gid = lax.broadcasted_iota(jnp.int32, (2 * NBINS, HIST), 1) // L
    bid = lax.broadcasted_iota(jnp.int32, (2 * NBINS, HIST), 0)
    colb = jnp.broadcast_to(col, (2 * NBINS, HIST))
    masked = jnp.where(gid == bid, colb, 0.0)
    o_ref[...] = jnp.sum(masked, axis=1, keepdims=True)


_red_call = pl.pallas_call(
    _red_body,
    out_shape=jax.ShapeDtypeStruct((2 * NBINS, 1), jnp.float32),
)


def kernel(pred, target):
    parts = _hist_call(pred, target)
    h = jnp.reshape(parts, (NW, HIST))
    counts = _red_call(h)[:, 0]
    p = counts[0:NBINS]
    t = counts[NBINS:2 * NBINS]
    p = p / p.sum()
    t = t / t.sum()
    p = p + 1e-08
    t = t + 1e-08
    return jnp.sum(t * (jnp.log(t) - jnp.log(p))) / NBINS
